# Initial kernel scaffold; baseline (speedup 1.0000x reference)
import jax
import jax.numpy as jnp
from jax.experimental import pallas as pl


def kernel(x, edge_attr, W, b, edge_index, time_steps):
    raise NotImplementedError("write your pallas kernel here")


# speedup vs baseline: 16.6801x; 16.6801x over previous
import functools

import jax
import jax.numpy as jnp
from jax import lax
from jax.experimental import pallas as pl
from jax.experimental.pallas import tpu as pltpu
from jax.experimental.pallas import tpu_sc as plsc

ACC_SCALE = 0.02
MAX_VEL = 0.1

CH = 128
NC, NS = 2, 16
NW = NC * NS


def _pick_cb(rpw):
    for cb in (142, 120, 96, 71, 62, 48, 32, 24, 16, 8, 4, 2, 1):
        if rpw % cb == 0:
            return cb
    return 1


def _sc_hist(src2, dst2, attr2x, zn):
    R = src2.shape[0]
    N = zn.shape[0]
    RPW = R // NW
    REM = R - RPW * NW
    CB = _pick_cb(RPW)
    NB = RPW // CB

    mesh = plsc.VectorSubcoreMesh(core_axis_name="c", subcore_axis_name="s")

    @functools.partial(
        pl.kernel,
        out_type=[
            jax.ShapeDtypeStruct((NC, N), jnp.float32),
            jax.ShapeDtypeStruct((NC, N), jnp.float32),
            jax.ShapeDtypeStruct((NC, N), jnp.float32),
        ],
        mesh=mesh,
        compiler_params=pltpu.CompilerParams(use_tc_tiling_on_sc=False, needs_layout_passes=False),
        scratch_types=[
            pltpu.VMEM_SHARED((N,), jnp.float32),
            pltpu.VMEM_SHARED((N,), jnp.float32),
            pltpu.VMEM_SHARED((N,), jnp.float32),
            pltpu.VMEM((CB, CH), jnp.int32),
            pltpu.VMEM((CB, CH), jnp.int32),
            pltpu.VMEM((CB, 2 * CH), jnp.float32),
            pltpu.VMEM((2, CH), jnp.float32),
            pltpu.VMEM((CH,), jnp.float32),
            pltpu.SemaphoreType.DMA,
        ],
    )
    def hist(src_h, dst_h, attr_h, zn_h, deg_o, epn_o, cda_o,
             deg_s, epn_s, cda_s, sbuf, dbuf, lbuf, cbuf, ones_v, sem):
        c = lax.axis_index("c")
        s = lax.axis_index("s")
        wid = s * NC + c

        for j in range(CH // 16):
            ones_v[pl.ds(j * 16, 16)] = jnp.ones((16,), jnp.float32)

        @pl.when(s == 0)
        def _():
            pltpu.sync_copy(zn_h, deg_s)

        @pl.when(s == 1)
        def _():
            pltpu.sync_copy(zn_h, epn_s)

        @pl.when(s == 2)
        def _():
            pltpu.sync_copy(zn_h, cda_s)

        plsc.subcore_barrier()

        odd = lax.iota(jnp.int32, 16) * 2 + 1

        def do_chunk(k, slot):
            rowbase = jnp.full((16,), k, jnp.int32)
            for j in range(CH // 16):
                vals = plsc.load_gather(lbuf, [rowbase, odd + (32 * j)])
                cbuf[slot, pl.ds(j * 16, 16)] = vals
            d0 = pltpu.async_copy(ones_v, deg_s.at[dbuf.at[k]], sem, add=True)
            d1 = pltpu.async_copy(ones_v, epn_s.at[sbuf.at[k]], sem, add=True)
            d2 = pltpu.async_copy(cbuf.at[slot], cda_s.at[sbuf.at[k]], sem,
                                  add=True)
            return d0, d1, d2

        def block(blk, _):
            row0 = wid * RPW + blk * CB
            pltpu.sync_copy(src_h.at[pl.ds(row0, CB)], sbuf)
            pltpu.sync_copy(dst_h.at[pl.ds(row0, CB)], dbuf)
            pltpu.sync_copy(attr_h.at[pl.ds(row0, CB)], lbuf)

            def inner(j, _):
                a = do_chunk(2 * j, 0)
                b_ = do_chunk(2 * j + 1, 1)
                for d in a + b_:
                    d.wait()
                return 0

            lax.fori_loop(0, CB // 2, inner, 0)
            if CB % 2:
                for d in do_chunk(CB - 1, 0):
                    d.wait()
            return 0

        lax.fori_loop(0, NB, block, 0)

        if REM > 0:
            @pl.when((c == 0) & (s < REM))
            def _():
                rrow = NW * RPW + s
                pltpu.sync_copy(src_h.at[pl.ds(rrow, 1)], sbuf.at[pl.ds(0, 1)])
                pltpu.sync_copy(dst_h.at[pl.ds(rrow, 1)], dbuf.at[pl.ds(0, 1)])
                pltpu.sync_copy(attr_h.at[pl.ds(rrow, 1)],
                                lbuf.at[pl.ds(0, 1)])
                for d in do_chunk(0, 0):
                    d.wait()

        plsc.subcore_barrier()

        @pl.when(s == 0)
        def _():
            pltpu.sync_copy(deg_s, deg_o.at[c])

        @pl.when(s == 1)
        def _():
            pltpu.sync_copy(epn_s, epn_o.at[c])

        @pl.when(s == 2)
        def _():
            pltpu.sync_copy(cda_s, cda_o.at[c])

    return hist(src2, dst2, attr2x, zn)


def _sc_msg(src2, dst2, xs0, xs1, zn):
    R = src2.shape[0]
    N = xs0.shape[0]
    RPW = R // NW
    REM = R - RPW * NW
    CB = _pick_cb(RPW)
    NB = RPW // CB

    mesh = plsc.VectorSubcoreMesh(core_axis_name="c", subcore_axis_name="s")

    @functools.partial(
        pl.kernel,
        out_type=[
            jax.ShapeDtypeStruct((NC, N), jnp.float32),
            jax.ShapeDtypeStruct((NC, N), jnp.float32),
        ],
        mesh=mesh,
        compiler_params=pltpu.CompilerParams(use_tc_tiling_on_sc=False, needs_layout_passes=False),
        scratch_types=[
            pltpu.VMEM_SHARED((N,), jnp.float32),
            pltpu.VMEM_SHARED((N,), jnp.float32),
            pltpu.VMEM_SHARED((N,), jnp.float32),
            pltpu.VMEM_SHARED((N,), jnp.float32),
            pltpu.VMEM((CB, CH), jnp.int32),
            pltpu.VMEM((CB, CH), jnp.int32),
            pltpu.VMEM((CB, CH), jnp.float32),
            pltpu.VMEM((CB, CH), jnp.float32),
            pltpu.SemaphoreType.DMA,
            pltpu.SemaphoreType.DMA,
        ],
    )
    def msg(src_h, dst_h, xs0_h, xs1_h, zn_h, a0_o, a1_o,
            xs0_s, xs1_s, a0_s, a1_s, sbuf, dbuf, m0, m1, gsem, ssem):
        c = lax.axis_index("c")
        s = lax.axis_index("s")
        wid = s * NC + c

        @pl.when(s == 0)
        def _():
            pltpu.sync_copy(xs0_h, xs0_s)

        @pl.when(s == 1)
        def _():
            pltpu.sync_copy(xs1_h, xs1_s)

        @pl.when(s == 2)
        def _():
            pltpu.sync_copy(zn_h, a0_s)

        @pl.when(s == 3)
        def _():
            pltpu.sync_copy(zn_h, a1_s)

        plsc.subcore_barrier()

        def do_chunk(k):
            g0 = pltpu.async_copy(xs0_s.at[sbuf.at[k]], m0.at[k], gsem)
            g1 = pltpu.async_copy(xs1_s.at[sbuf.at[k]], m1.at[k], gsem)
            g0.wait()
            s0 = pltpu.async_copy(m0.at[k], a0_s.at[dbuf.at[k]], ssem,
                                  add=True)
            g1.wait()
            s1 = pltpu.async_copy(m1.at[k], a1_s.at[dbuf.at[k]], ssem,
                                  add=True)
            return s0, s1

        def block(blk, _):
            row0 = wid * RPW + blk * CB
            pltpu.sync_copy(src_h.at[pl.ds(row0, CB)], sbuf)
            pltpu.sync_copy(dst_h.at[pl.ds(row0, CB)], dbuf)

            def inner(j, _):
                a = do_chunk(2 * j)
                b_ = do_chunk(2 * j + 1)
                for d in a + b_:
                    d.wait()
                return 0

            lax.fori_loop(0, CB // 2, inner, 0)
            if CB % 2:
                for d in do_chunk(CB - 1):
                    d.wait()
            return 0

        lax.fori_loop(0, NB, block, 0)

        if REM > 0:
            @pl.when((c == 0) & (s < REM))
            def _():
                rrow = NW * RPW + s
                pltpu.sync_copy(src_h.at[pl.ds(rrow, 1)], sbuf.at[pl.ds(0, 1)])
                pltpu.sync_copy(dst_h.at[pl.ds(rrow, 1)], dbuf.at[pl.ds(0, 1)])
                for d in do_chunk(0):
                    d.wait()

        plsc.subcore_barrier()

        @pl.when(s == 0)
        def _():
            pltpu.sync_copy(a0_s, a0_o.at[c])

        @pl.when(s == 1)
        def _():
            pltpu.sync_copy(a1_s, a1_o.at[c])

    return msg(src2, dst2, xs0, xs1, zn)


def _tc_mid(xt, W, degp):
    _, M, L = xt.shape

    def body(xt_ref, w_ref, degp_ref, xs_ref, dinv_ref):
        deg = degp_ref[0] + degp_ref[1] + 1.0
        dinv = lax.rsqrt(deg)
        xw0 = (xt_ref[0] * w_ref[0, 0] + xt_ref[1] * w_ref[1, 0]
               + xt_ref[2] * w_ref[2, 0] + xt_ref[3] * w_ref[3, 0]
               + xt_ref[4] * w_ref[4, 0])
        xw1 = (xt_ref[0] * w_ref[0, 1] + xt_ref[1] * w_ref[1, 1]
               + xt_ref[2] * w_ref[2, 1] + xt_ref[3] * w_ref[3, 1]
               + xt_ref[4] * w_ref[4, 1])
        xs_ref[0] = xw0 * dinv
        xs_ref[1] = xw1 * dinv
        dinv_ref[...] = dinv

    return pl.pallas_call(
        body,
        out_shape=[
            jax.ShapeDtypeStruct((2, M, L), jnp.float32),
            jax.ShapeDtypeStruct((M, L), jnp.float32),
        ],
        in_specs=[
            pl.BlockSpec(memory_space=pltpu.VMEM),
            pl.BlockSpec(memory_space=pltpu.SMEM),
            pl.BlockSpec(memory_space=pltpu.VMEM),
        ],
    )(xt, W, degp)


def _tc_final(xt, dinv, xs2, a0p, a1p, b, epnp, cdp):
    _, M, L = xt.shape
    n_nodes = M * L

    def body(xt_ref, dinv_ref, xs_ref, a0_ref, a1_ref, b_ref, epnp_ref,
             cdp_ref, nx_ref, sc_ref):
        dinv = dinv_ref[...]
        x4 = xt_ref[4]
        acc0 = a0_ref[0] + a0_ref[1] + xs_ref[0]
        acc1 = a1_ref[0] + a1_ref[1] + xs_ref[1]
        g0 = dinv * acc0 + b_ref[0]
        g1 = dinv * acc1 + b_ref[1]
        food = (x4 == 1.0).astype(jnp.float32)
        a0 = g0 * ACC_SCALE * food
        a1 = g1 * ACC_SCALE * food
        v0 = jnp.clip(xt_ref[2] + a0, -MAX_VEL, MAX_VEL)
        v1 = jnp.clip(xt_ref[3] + a1, -MAX_VEL, MAX_VEL)
        p0 = xt_ref[0] + v0
        p1 = xt_ref[1] + v1
        nx_ref[0] = p0
        nx_ref[1] = p1
        nx_ref[2] = v0
        nx_ref[3] = v1
        nx_ref[4] = x4

        inv_n = jnp.float32(1.0 / n_nodes)
        vb0 = jnp.sum(jnp.abs(v0)) * inv_n
        vb1 = jnp.sum(jnp.abs(v1)) * inv_n
        pp0 = jnp.sum(jnp.abs(p0)) * inv_n
        pp1 = jnp.sum(jnp.abs(p1)) * inv_n
        ap0 = jnp.abs(p0)
        ap1 = jnp.abs(p1)
        border = (jnp.sum(jnp.log(ap0 + 1e-12) * (ap0 > 1.0))
                  + jnp.sum(jnp.log(ap1 + 1e-12) * (ap1 > 1.0)))
        epn = epnp_ref[0] + epnp_ref[1]
        fr = jnp.sum(((epn > 4.0) & (x4 == 0.0)).astype(jnp.float32))
        cd = cdp_ref[0] + cdp_ref[1]
        dc = jnp.sum(((x4 == 1.0) & (cd == 0.0)).astype(jnp.float32))

        idx = lax.broadcasted_iota(jnp.int32, (1, 128), 1)
        row = (jnp.where(idx == 0, vb0, 0.0) + jnp.where(idx == 1, vb1, 0.0)
               + jnp.where(idx == 2, pp0, 0.0) + jnp.where(idx == 3, pp1, 0.0)
               + jnp.where(idx == 4, border, 0.0)
               + jnp.where(idx == 5, fr, 0.0) + jnp.where(idx == 6, dc, 0.0))
        sc_ref[...] = row

    return pl.pallas_call(
        body,
        out_shape=[
            jax.ShapeDtypeStruct((5, M, L), jnp.float32),
            jax.ShapeDtypeStruct((1, 128), jnp.float32),
        ],
        in_specs=[
            pl.BlockSpec(memory_space=pltpu.VMEM),
            pl.BlockSpec(memory_space=pltpu.VMEM),
            pl.BlockSpec(memory_space=pltpu.VMEM),
            pl.BlockSpec(memory_space=pltpu.VMEM),
            pl.BlockSpec(memory_space=pltpu.VMEM),
            pl.BlockSpec(memory_space=pltpu.SMEM),
            pl.BlockSpec(memory_space=pltpu.VMEM),
            pl.BlockSpec(memory_space=pltpu.VMEM),
        ],
    )(xt, dinv, xs2, a0p, a1p, b, epnp, cdp)


def kernel(x, edge_attr, W, b, edge_index, time_steps=1):
    N = x.shape[0]
    E = edge_index.shape[1]
    M, L = 800, 125
    if M * L != N:
        M, L = N // 8, 8

    R = E // CH
    src2 = edge_index[0].reshape(R, CH)
    dst2 = edge_index[1].reshape(R, CH)
    attr2x = edge_attr.reshape(R, 2 * CH)
    zn = jnp.zeros((N,), jnp.float32)

    deg_p, epn_p, cda_p = _sc_hist(src2, dst2, attr2x, zn)
    degp = deg_p.reshape(NC, M, L)
    epnp = epn_p.reshape(NC, M, L)
    cdp = cda_p.reshape(NC, M, L)

    xt0 = x.T.reshape(5, M, L)

    def body(_, carry):
        xt, vb, pp, bc, fr, dc = carry
        xs2, dinv = _tc_mid(xt, W, degp)
        xsf = xs2.reshape(2, N)
        a0_p, a1_p = _sc_msg(src2, dst2, xsf[0], xsf[1], zn)
        a0p = a0_p.reshape(NC, M, L)
        a1p = a1_p.reshape(NC, M, L)
        nxt, scal = _tc_final(xt, dinv, xs2, a0p, a1p, b, epnp, cdp)
        s = scal[0]
        return (nxt, vb + s[0:2], pp + s[2:4], bc + s[4], fr + s[5], dc + s[6])

    carry = (xt0, jnp.zeros((2,), jnp.float32), jnp.zeros((2,), jnp.float32),
             jnp.float32(0.0), jnp.float32(0.0), jnp.float32(0.0))
    xt, vb, pp, bc, fr, dc = lax.fori_loop(0, time_steps, body, carry)
    x_out = xt.reshape(5, N).T
    return (x_out, vb, pp, bc, fr, dc)


# speedup vs baseline: 171.7574x; 10.2971x over previous
import functools

import jax
import jax.numpy as jnp
from jax import lax
from jax.experimental import pallas as pl
from jax.experimental.pallas import tpu as pltpu
from jax.experimental.pallas import tpu_sc as plsc

ACC_SCALE = 0.02
MAX_VEL = 0.1

CH = 128
NC, NS = 2, 16
NW = NC * NS


def _pick_cb(rpw):
    for cb in (142, 120, 96, 71, 62, 48, 32, 24, 16, 8, 4, 2, 1):
        if rpw % cb == 0:
            return cb
    return 1


def _sc_hist(src2, dst2, attr1, zn):
    R = src2.shape[0]
    N = zn.shape[0]
    RPW = R // NW
    REM = R - RPW * NW
    CB = _pick_cb(RPW)
    NB = RPW // CB

    mesh = plsc.VectorSubcoreMesh(core_axis_name="c", subcore_axis_name="s")

    @functools.partial(
        pl.kernel,
        out_type=[
            jax.ShapeDtypeStruct((NC, N), jnp.float32),
            jax.ShapeDtypeStruct((NC, N), jnp.float32),
            jax.ShapeDtypeStruct((NC, N), jnp.float32),
        ],
        mesh=mesh,
        compiler_params=pltpu.CompilerParams(use_tc_tiling_on_sc=False, needs_layout_passes=False),
        scratch_types=[
            pltpu.VMEM_SHARED((N,), jnp.float32),
            pltpu.VMEM_SHARED((N,), jnp.float32),
            pltpu.VMEM_SHARED((N,), jnp.float32),
            pltpu.VMEM((CB, CH), jnp.int32),
            pltpu.VMEM((CB, CH), jnp.int32),
            pltpu.VMEM((CB, CH), jnp.float32),
            pltpu.VMEM((CH,), jnp.float32),
            pltpu.SemaphoreType.DMA,
        ],
    )
    def hist(src_h, dst_h, attr_h, zn_h, deg_o, epn_o, cda_o,
             deg_s, epn_s, cda_s, sbuf, dbuf, abuf, ones_v, sem):
        c = lax.axis_index("c")
        s = lax.axis_index("s")
        wid = s * NC + c

        for j in range(CH // 16):
            ones_v[pl.ds(j * 16, 16)] = jnp.ones((16,), jnp.float32)

        @pl.when(s == 0)
        def _():
            pltpu.sync_copy(zn_h, deg_s)

        @pl.when(s == 1)
        def _():
            pltpu.sync_copy(zn_h, epn_s)

        @pl.when(s == 2)
        def _():
            pltpu.sync_copy(zn_h, cda_s)

        plsc.subcore_barrier()

        def do_chunk(k):
            d0 = pltpu.async_copy(ones_v, deg_s.at[dbuf.at[k]], sem, add=True)
            d1 = pltpu.async_copy(ones_v, epn_s.at[sbuf.at[k]], sem, add=True)
            d2 = pltpu.async_copy(abuf.at[k], cda_s.at[sbuf.at[k]], sem,
                                  add=True)
            return d0, d1, d2

        def block(blk, _):
            row0 = wid * RPW + blk * CB
            pltpu.sync_copy(src_h.at[pl.ds(row0, CB)], sbuf)
            pltpu.sync_copy(dst_h.at[pl.ds(row0, CB)], dbuf)
            pltpu.sync_copy(attr_h.at[pl.ds(row0, CB)], abuf)

            def inner(j, _):
                a = do_chunk(2 * j)
                b_ = do_chunk(2 * j + 1)
                for d in a + b_:
                    d.wait()
                return 0

            lax.fori_loop(0, CB // 2, inner, 0)
            if CB % 2:
                for d in do_chunk(CB - 1):
                    d.wait()
            return 0

        lax.fori_loop(0, NB, block, 0)

        if REM > 0:
            @pl.when((c == 0) & (s < REM))
            def _():
                rrow = NW * RPW + s
                pltpu.sync_copy(src_h.at[pl.ds(rrow, 1)], sbuf.at[pl.ds(0, 1)])
                pltpu.sync_copy(dst_h.at[pl.ds(rrow, 1)], dbuf.at[pl.ds(0, 1)])
                pltpu.sync_copy(attr_h.at[pl.ds(rrow, 1)],
                                abuf.at[pl.ds(0, 1)])
                for d in do_chunk(0):
                    d.wait()

        plsc.subcore_barrier()

        @pl.when(s == 0)
        def _():
            pltpu.sync_copy(deg_s, deg_o.at[c])

        @pl.when(s == 1)
        def _():
            pltpu.sync_copy(epn_s, epn_o.at[c])

        @pl.when(s == 2)
        def _():
            pltpu.sync_copy(cda_s, cda_o.at[c])

    return hist(src2, dst2, attr1, zn)


def _sc_msg(src2, dst2, xs0, xs1, zn):
    R = src2.shape[0]
    N = xs0.shape[0]
    RPW = R // NW
    REM = R - RPW * NW
    CB = _pick_cb(RPW)
    NB = RPW // CB

    mesh = plsc.VectorSubcoreMesh(core_axis_name="c", subcore_axis_name="s")

    @functools.partial(
        pl.kernel,
        out_type=[
            jax.ShapeDtypeStruct((NC, N), jnp.float32),
            jax.ShapeDtypeStruct((NC, N), jnp.float32),
        ],
        mesh=mesh,
        compiler_params=pltpu.CompilerParams(use_tc_tiling_on_sc=False, needs_layout_passes=False),
        scratch_types=[
            pltpu.VMEM_SHARED((N,), jnp.float32),
            pltpu.VMEM_SHARED((N,), jnp.float32),
            pltpu.VMEM_SHARED((N,), jnp.float32),
            pltpu.VMEM_SHARED((N,), jnp.float32),
            pltpu.VMEM((CB, CH), jnp.int32),
            pltpu.VMEM((CB, CH), jnp.int32),
            pltpu.VMEM((CB, CH), jnp.float32),
            pltpu.VMEM((CB, CH), jnp.float32),
            pltpu.SemaphoreType.DMA,
            pltpu.SemaphoreType.DMA,
        ],
    )
    def msg(src_h, dst_h, xs0_h, xs1_h, zn_h, a0_o, a1_o,
            xs0_s, xs1_s, a0_s, a1_s, sbuf, dbuf, m0, m1, gsem, ssem):
        c = lax.axis_index("c")
        s = lax.axis_index("s")
        wid = s * NC + c

        @pl.when(s == 0)
        def _():
            pltpu.sync_copy(xs0_h, xs0_s)

        @pl.when(s == 1)
        def _():
            pltpu.sync_copy(xs1_h, xs1_s)

        @pl.when(s == 2)
        def _():
            pltpu.sync_copy(zn_h, a0_s)

        @pl.when(s == 3)
        def _():
            pltpu.sync_copy(zn_h, a1_s)

        plsc.subcore_barrier()

        def do_chunk(k):
            g0 = pltpu.async_copy(xs0_s.at[sbuf.at[k]], m0.at[k], gsem)
            g1 = pltpu.async_copy(xs1_s.at[sbuf.at[k]], m1.at[k], gsem)
            g0.wait()
            s0 = pltpu.async_copy(m0.at[k], a0_s.at[dbuf.at[k]], ssem,
                                  add=True)
            g1.wait()
            s1 = pltpu.async_copy(m1.at[k], a1_s.at[dbuf.at[k]], ssem,
                                  add=True)
            return s0, s1

        def block(blk, _):
            row0 = wid * RPW + blk * CB
            pltpu.sync_copy(src_h.at[pl.ds(row0, CB)], sbuf)
            pltpu.sync_copy(dst_h.at[pl.ds(row0, CB)], dbuf)

            def inner(j, _):
                a = do_chunk(2 * j)
                b_ = do_chunk(2 * j + 1)
                for d in a + b_:
                    d.wait()
                return 0

            lax.fori_loop(0, CB // 2, inner, 0)
            if CB % 2:
                for d in do_chunk(CB - 1):
                    d.wait()
            return 0

        lax.fori_loop(0, NB, block, 0)

        if REM > 0:
            @pl.when((c == 0) & (s < REM))
            def _():
                rrow = NW * RPW + s
                pltpu.sync_copy(src_h.at[pl.ds(rrow, 1)], sbuf.at[pl.ds(0, 1)])
                pltpu.sync_copy(dst_h.at[pl.ds(rrow, 1)], dbuf.at[pl.ds(0, 1)])
                for d in do_chunk(0):
                    d.wait()

        plsc.subcore_barrier()

        @pl.when(s == 0)
        def _():
            pltpu.sync_copy(a0_s, a0_o.at[c])

        @pl.when(s == 1)
        def _():
            pltpu.sync_copy(a1_s, a1_o.at[c])

    return msg(src2, dst2, xs0, xs1, zn)


def _tc_mid(xt, W, degp):
    _, M, L = xt.shape

    def body(xt_ref, w_ref, degp_ref, xs_ref, dinv_ref):
        deg = degp_ref[0] + degp_ref[1] + 1.0
        dinv = lax.rsqrt(deg)
        xw0 = (xt_ref[0] * w_ref[0, 0] + xt_ref[1] * w_ref[1, 0]
               + xt_ref[2] * w_ref[2, 0] + xt_ref[3] * w_ref[3, 0]
               + xt_ref[4] * w_ref[4, 0])
        xw1 = (xt_ref[0] * w_ref[0, 1] + xt_ref[1] * w_ref[1, 1]
               + xt_ref[2] * w_ref[2, 1] + xt_ref[3] * w_ref[3, 1]
               + xt_ref[4] * w_ref[4, 1])
        xs_ref[0] = xw0 * dinv
        xs_ref[1] = xw1 * dinv
        dinv_ref[...] = dinv

    return pl.pallas_call(
        body,
        out_shape=[
            jax.ShapeDtypeStruct((2, M, L), jnp.float32),
            jax.ShapeDtypeStruct((M, L), jnp.float32),
        ],
        in_specs=[
            pl.BlockSpec(memory_space=pltpu.VMEM),
            pl.BlockSpec(memory_space=pltpu.SMEM),
            pl.BlockSpec(memory_space=pltpu.VMEM),
        ],
    )(xt, W, degp)


def _tc_final(xt, dinv, xs2, a0p, a1p, b, epnp, cdp):
    _, M, L = xt.shape
    n_nodes = M * L

    def body(xt_ref, dinv_ref, xs_ref, a0_ref, a1_ref, b_ref, epnp_ref,
             cdp_ref, nx_ref, sc_ref):
        dinv = dinv_ref[...]
        x4 = xt_ref[4]
        acc0 = a0_ref[0] + a0_ref[1] + xs_ref[0]
        acc1 = a1_ref[0] + a1_ref[1] + xs_ref[1]
        g0 = dinv * acc0 + b_ref[0]
        g1 = dinv * acc1 + b_ref[1]
        food = (x4 == 1.0).astype(jnp.float32)
        a0 = g0 * ACC_SCALE * food
        a1 = g1 * ACC_SCALE * food
        v0 = jnp.clip(xt_ref[2] + a0, -MAX_VEL, MAX_VEL)
        v1 = jnp.clip(xt_ref[3] + a1, -MAX_VEL, MAX_VEL)
        p0 = xt_ref[0] + v0
        p1 = xt_ref[1] + v1
        nx_ref[0] = p0
        nx_ref[1] = p1
        nx_ref[2] = v0
        nx_ref[3] = v1
        nx_ref[4] = x4

        inv_n = jnp.float32(1.0 / n_nodes)
        vb0 = jnp.sum(jnp.abs(v0)) * inv_n
        vb1 = jnp.sum(jnp.abs(v1)) * inv_n
        pp0 = jnp.sum(jnp.abs(p0)) * inv_n
        pp1 = jnp.sum(jnp.abs(p1)) * inv_n
        ap0 = jnp.abs(p0)
        ap1 = jnp.abs(p1)
        border = (jnp.sum(jnp.log(ap0 + 1e-12) * (ap0 > 1.0))
                  + jnp.sum(jnp.log(ap1 + 1e-12) * (ap1 > 1.0)))
        epn = epnp_ref[0] + epnp_ref[1]
        fr = jnp.sum(((epn > 4.0) & (x4 == 0.0)).astype(jnp.float32))
        cd = cdp_ref[0] + cdp_ref[1]
        dc = jnp.sum(((x4 == 1.0) & (cd == 0.0)).astype(jnp.float32))

        idx = lax.broadcasted_iota(jnp.int32, (1, 128), 1)
        row = (jnp.where(idx == 0, vb0, 0.0) + jnp.where(idx == 1, vb1, 0.0)
               + jnp.where(idx == 2, pp0, 0.0) + jnp.where(idx == 3, pp1, 0.0)
               + jnp.where(idx == 4, border, 0.0)
               + jnp.where(idx == 5, fr, 0.0) + jnp.where(idx == 6, dc, 0.0))
        sc_ref[...] = row

    return pl.pallas_call(
        body,
        out_shape=[
            jax.ShapeDtypeStruct((5, M, L), jnp.float32),
            jax.ShapeDtypeStruct((1, 128), jnp.float32),
        ],
        in_specs=[
            pl.BlockSpec(memory_space=pltpu.VMEM),
            pl.BlockSpec(memory_space=pltpu.VMEM),
            pl.BlockSpec(memory_space=pltpu.VMEM),
            pl.BlockSpec(memory_space=pltpu.VMEM),
            pl.BlockSpec(memory_space=pltpu.VMEM),
            pl.BlockSpec(memory_space=pltpu.SMEM),
            pl.BlockSpec(memory_space=pltpu.VMEM),
            pl.BlockSpec(memory_space=pltpu.VMEM),
        ],
    )(xt, dinv, xs2, a0p, a1p, b, epnp, cdp)


def kernel(x, edge_attr, W, b, edge_index, time_steps=1):
    N = x.shape[0]
    E = edge_index.shape[1]
    M, L = 800, 125
    if M * L != N:
        M, L = N // 8, 8

    R = E // CH
    src2 = edge_index[0].reshape(R, CH)
    dst2 = edge_index[1].reshape(R, CH)
    attr1r = edge_attr[:, 1].reshape(R, CH)
    zn = jnp.zeros((N,), jnp.float32)

    deg_p, epn_p, cda_p = _sc_hist(src2, dst2, attr1r, zn)
    degp = deg_p.reshape(NC, M, L)
    epnp = epn_p.reshape(NC, M, L)
    cdp = cda_p.reshape(NC, M, L)

    xt0 = x.T.reshape(5, M, L)

    def body(_, carry):
        xt, vb, pp, bc, fr, dc = carry
        xs2, dinv = _tc_mid(xt, W, degp)
        xsf = xs2.reshape(2, N)
        a0_p, a1_p = _sc_msg(src2, dst2, xsf[0], xsf[1], zn)
        a0p = a0_p.reshape(NC, M, L)
        a1p = a1_p.reshape(NC, M, L)
        nxt, scal = _tc_final(xt, dinv, xs2, a0p, a1p, b, epnp, cdp)
        s = scal[0]
        return (nxt, vb + s[0:2], pp + s[2:4], bc + s[4], fr + s[5], dc + s[6])

    carry = (xt0, jnp.zeros((2,), jnp.float32), jnp.zeros((2,), jnp.float32),
             jnp.float32(0.0), jnp.float32(0.0), jnp.float32(0.0))
    xt, vb, pp, bc, fr, dc = lax.fori_loop(0, time_steps, body, carry)
    x_out = xt.reshape(5, N).T
    return (x_out, vb, pp, bc, fr, dc)


# speedup vs baseline: 198.7465x; 1.1571x over previous
import functools

import jax
import jax.numpy as jnp
from jax import lax
from jax.experimental import pallas as pl
from jax.experimental.pallas import tpu as pltpu
from jax.experimental.pallas import tpu_sc as plsc

ACC_SCALE = 0.02
MAX_VEL = 0.1

CH = 128
NC, NS = 2, 16
NW = NC * NS
CB = 64
G = 16


def _split(rows):
    main = (rows // (NW * CB)) * CB
    rem_total = rows - NW * main
    rem = rem_total // NW
    tail = rem_total - NW * rem
    return main, rem, tail


def _sc_hist(src2, dst2, attr1, zn):
    R = src2.shape[0]
    N = zn.shape[0]
    MAIN, REM, TAIL = _split(R)
    NB = MAIN // CB

    mesh = plsc.VectorSubcoreMesh(core_axis_name="c", subcore_axis_name="s")

    @functools.partial(
        pl.kernel,
        out_type=[
            jax.ShapeDtypeStruct((NC, N), jnp.float32),
            jax.ShapeDtypeStruct((NC, N), jnp.float32),
            jax.ShapeDtypeStruct((NC, N), jnp.float32),
        ],
        mesh=mesh,
        compiler_params=pltpu.CompilerParams(use_tc_tiling_on_sc=False,
                                             needs_layout_passes=False),
        scratch_types=[
            pltpu.VMEM_SHARED((N,), jnp.float32),
            pltpu.VMEM_SHARED((N,), jnp.float32),
            pltpu.VMEM_SHARED((N,), jnp.float32),
            pltpu.VMEM((CB, CH), jnp.int32),
            pltpu.VMEM((CB, CH), jnp.int32),
            pltpu.VMEM((CB, CH), jnp.float32),
            pltpu.VMEM((CH,), jnp.float32),
            pltpu.SemaphoreType.DMA,
        ],
    )
    def hist(src_h, dst_h, attr_h, zn_h, deg_o, epn_o, cda_o,
             deg_s, epn_s, cda_s, sbuf, dbuf, abuf, ones_v, sem):
        c = lax.axis_index("c")
        s = lax.axis_index("s")
        wid = s * NC + c

        for j in range(CH // 16):
            ones_v[pl.ds(j * 16, 16)] = jnp.ones((16,), jnp.float32)

        @pl.when(s == 0)
        def _():
            pltpu.sync_copy(zn_h, deg_s)

        @pl.when(s == 1)
        def _():
            pltpu.sync_copy(zn_h, epn_s)

        @pl.when(s == 2)
        def _():
            pltpu.sync_copy(zn_h, cda_s)

        plsc.subcore_barrier()

        def fire(k):
            pltpu.async_copy(ones_v, deg_s.at[dbuf.at[k]], sem, add=True)
            pltpu.async_copy(ones_v, epn_s.at[sbuf.at[k]], sem, add=True)
            pltpu.async_copy(abuf.at[k], cda_s.at[sbuf.at[k]], sem, add=True)

        def drain(k):
            pltpu.make_async_copy(ones_v, deg_s.at[dbuf.at[k]], sem).wait()
            pltpu.make_async_copy(ones_v, epn_s.at[sbuf.at[k]], sem).wait()
            pltpu.make_async_copy(abuf.at[k], cda_s.at[sbuf.at[k]], sem).wait()

        def do_span(r, n):
            pltpu.sync_copy(src_h.at[pl.ds(r, n)], sbuf.at[pl.ds(0, n)])
            pltpu.sync_copy(dst_h.at[pl.ds(r, n)], dbuf.at[pl.ds(0, n)])
            pltpu.sync_copy(attr_h.at[pl.ds(r, n)], abuf.at[pl.ds(0, n)])
            for k in range(n):
                fire(k)
            for k in range(n):
                drain(k)

        def blockm(blk, _):
            do_span(wid * MAIN + blk * CB, CB)
            return 0

        lax.fori_loop(0, NB, blockm, 0)

        if REM > 0:
            do_span(NW * MAIN + wid * REM, REM)

        if TAIL > 0:
            @pl.when((c == 0) & (s < TAIL))
            def _():
                do_span(NW * (MAIN + REM) + s, 1)

        plsc.subcore_barrier()

        @pl.when(s == 0)
        def _():
            pltpu.sync_copy(deg_s, deg_o.at[c])

        @pl.when(s == 1)
        def _():
            pltpu.sync_copy(epn_s, epn_o.at[c])

        @pl.when(s == 2)
        def _():
            pltpu.sync_copy(cda_s, cda_o.at[c])

    return hist(src2, dst2, attr1, zn)


def _sc_msg(src2, dst2, xs0, xs1, zn):
    R = src2.shape[0]
    N = xs0.shape[0]
    MAIN, REM, TAIL = _split(R)
    NB = MAIN // CB

    mesh = plsc.VectorSubcoreMesh(core_axis_name="c", subcore_axis_name="s")

    @functools.partial(
        pl.kernel,
        out_type=[
            jax.ShapeDtypeStruct((NC, N), jnp.float32),
            jax.ShapeDtypeStruct((NC, N), jnp.float32),
        ],
        mesh=mesh,
        compiler_params=pltpu.CompilerParams(use_tc_tiling_on_sc=False,
                                             needs_layout_passes=False),
        scratch_types=[
            pltpu.VMEM_SHARED((N,), jnp.float32),
            pltpu.VMEM_SHARED((N,), jnp.float32),
            pltpu.VMEM_SHARED((N,), jnp.float32),
            pltpu.VMEM_SHARED((N,), jnp.float32),
            pltpu.VMEM((CB, CH), jnp.int32),
            pltpu.VMEM((CB, CH), jnp.int32),
            pltpu.VMEM((CB, CH), jnp.float32),
            pltpu.VMEM((CB, CH), jnp.float32),
            pltpu.SemaphoreType.DMA,
            pltpu.SemaphoreType.DMA,
        ],
    )
    def msg(src_h, dst_h, xs0_h, xs1_h, zn_h, a0_o, a1_o,
            xs0_s, xs1_s, a0_s, a1_s, sbuf, dbuf, m0, m1, gsem, ssem):
        c = lax.axis_index("c")
        s = lax.axis_index("s")
        wid = s * NC + c

        @pl.when(s == 0)
        def _():
            pltpu.sync_copy(xs0_h, xs0_s)

        @pl.when(s == 1)
        def _():
            pltpu.sync_copy(xs1_h, xs1_s)

        @pl.when(s == 2)
        def _():
            pltpu.sync_copy(zn_h, a0_s)

        @pl.when(s == 3)
        def _():
            pltpu.sync_copy(zn_h, a1_s)

        plsc.subcore_barrier()

        def fire_gather(k):
            pltpu.async_copy(xs0_s.at[sbuf.at[k]], m0.at[k], gsem)
            pltpu.async_copy(xs1_s.at[sbuf.at[k]], m1.at[k], gsem)

        def drain_gather(k):
            pltpu.make_async_copy(xs0_s.at[sbuf.at[k]], m0.at[k], gsem).wait()
            pltpu.make_async_copy(xs1_s.at[sbuf.at[k]], m1.at[k], gsem).wait()

        def fire_scatter(k):
            pltpu.async_copy(m0.at[k], a0_s.at[dbuf.at[k]], ssem, add=True)
            pltpu.async_copy(m1.at[k], a1_s.at[dbuf.at[k]], ssem, add=True)

        def drain_scatter(k):
            pltpu.make_async_copy(m0.at[k], a0_s.at[dbuf.at[k]], ssem).wait()
            pltpu.make_async_copy(m1.at[k], a1_s.at[dbuf.at[k]], ssem).wait()

        def do_span(r, n, gsz):
            pltpu.sync_copy(src_h.at[pl.ds(r, n)], sbuf.at[pl.ds(0, n)])
            pltpu.sync_copy(dst_h.at[pl.ds(r, n)], dbuf.at[pl.ds(0, n)])
            for g0 in range(0, n, gsz):
                gn = min(gsz, n - g0)
                for t in range(gn):
                    fire_gather(g0 + t)
                for t in range(gn):
                    drain_gather(g0 + t)
                for t in range(gn):
                    fire_scatter(g0 + t)
            for k in range(n):
                drain_scatter(k)

        def blockm(blk, _):
            do_span(wid * MAIN + blk * CB, CB, G)
            return 0

        lax.fori_loop(0, NB, blockm, 0)

        if REM > 0:
            do_span(NW * MAIN + wid * REM, REM, G)

        if TAIL > 0:
            @pl.when((c == 0) & (s < TAIL))
            def _():
                do_span(NW * (MAIN + REM) + s, 1, 1)

        plsc.subcore_barrier()

        @pl.when(s == 0)
        def _():
            pltpu.sync_copy(a0_s, a0_o.at[c])

        @pl.when(s == 1)
        def _():
            pltpu.sync_copy(a1_s, a1_o.at[c])

    return msg(src2, dst2, xs0, xs1, zn)


def _tc_mid(xt, W, degp):
    _, M, L = xt.shape

    def body(xt_ref, w_ref, degp_ref, xs_ref, dinv_ref):
        deg = degp_ref[0] + degp_ref[1] + 1.0
        dinv = lax.rsqrt(deg)
        xw0 = (xt_ref[0] * w_ref[0, 0] + xt_ref[1] * w_ref[1, 0]
               + xt_ref[2] * w_ref[2, 0] + xt_ref[3] * w_ref[3, 0]
               + xt_ref[4] * w_ref[4, 0])
        xw1 = (xt_ref[0] * w_ref[0, 1] + xt_ref[1] * w_ref[1, 1]
               + xt_ref[2] * w_ref[2, 1] + xt_ref[3] * w_ref[3, 1]
               + xt_ref[4] * w_ref[4, 1])
        xs_ref[0] = xw0 * dinv
        xs_ref[1] = xw1 * dinv
        dinv_ref[...] = dinv

    return pl.pallas_call(
        body,
        out_shape=[
            jax.ShapeDtypeStruct((2, M, L), jnp.float32),
            jax.ShapeDtypeStruct((M, L), jnp.float32),
        ],
        in_specs=[
            pl.BlockSpec(memory_space=pltpu.VMEM),
            pl.BlockSpec(memory_space=pltpu.SMEM),
            pl.BlockSpec(memory_space=pltpu.VMEM),
        ],
    )(xt, W, degp)


def _tc_final(xt, dinv, xs2, a0p, a1p, b, epnp, cdp):
    _, M, L = xt.shape
    n_nodes = M * L

    def body(xt_ref, dinv_ref, xs_ref, a0_ref, a1_ref, b_ref, epnp_ref,
             cdp_ref, nx_ref, sc_ref):
        dinv = dinv_ref[...]
        x4 = xt_ref[4]
        acc0 = a0_ref[0] + a0_ref[1] + xs_ref[0]
        acc1 = a1_ref[0] + a1_ref[1] + xs_ref[1]
        g0 = dinv * acc0 + b_ref[0]
        g1 = dinv * acc1 + b_ref[1]
        food = (x4 == 1.0).astype(jnp.float32)
        a0 = g0 * ACC_SCALE * food
        a1 = g1 * ACC_SCALE * food
        v0 = jnp.clip(xt_ref[2] + a0, -MAX_VEL, MAX_VEL)
        v1 = jnp.clip(xt_ref[3] + a1, -MAX_VEL, MAX_VEL)
        p0 = xt_ref[0] + v0
        p1 = xt_ref[1] + v1
        nx_ref[0] = p0
        nx_ref[1] = p1
        nx_ref[2] = v0
        nx_ref[3] = v1
        nx_ref[4] = x4

        inv_n = jnp.float32(1.0 / n_nodes)
        vb0 = jnp.sum(jnp.abs(v0)) * inv_n
        vb1 = jnp.sum(jnp.abs(v1)) * inv_n
        pp0 = jnp.sum(jnp.abs(p0)) * inv_n
        pp1 = jnp.sum(jnp.abs(p1)) * inv_n
        ap0 = jnp.abs(p0)
        ap1 = jnp.abs(p1)
        border = (jnp.sum(jnp.log(ap0 + 1e-12) * (ap0 > 1.0))
                  + jnp.sum(jnp.log(ap1 + 1e-12) * (ap1 > 1.0)))
        epn = epnp_ref[0] + epnp_ref[1]
        fr = jnp.sum(((epn > 4.0) & (x4 == 0.0)).astype(jnp.float32))
        cd = cdp_ref[0] + cdp_ref[1]
        dc = jnp.sum(((x4 == 1.0) & (cd == 0.0)).astype(jnp.float32))

        idx = lax.broadcasted_iota(jnp.int32, (1, 128), 1)
        row = (jnp.where(idx == 0, vb0, 0.0) + jnp.where(idx == 1, vb1, 0.0)
               + jnp.where(idx == 2, pp0, 0.0) + jnp.where(idx == 3, pp1, 0.0)
               + jnp.where(idx == 4, border, 0.0)
               + jnp.where(idx == 5, fr, 0.0) + jnp.where(idx == 6, dc, 0.0))
        sc_ref[...] = row

    return pl.pallas_call(
        body,
        out_shape=[
            jax.ShapeDtypeStruct((5, M, L), jnp.float32),
            jax.ShapeDtypeStruct((1, 128), jnp.float32),
        ],
        in_specs=[
            pl.BlockSpec(memory_space=pltpu.VMEM),
            pl.BlockSpec(memory_space=pltpu.VMEM),
            pl.BlockSpec(memory_space=pltpu.VMEM),
            pl.BlockSpec(memory_space=pltpu.VMEM),
            pl.BlockSpec(memory_space=pltpu.VMEM),
            pl.BlockSpec(memory_space=pltpu.SMEM),
            pl.BlockSpec(memory_space=pltpu.VMEM),
            pl.BlockSpec(memory_space=pltpu.VMEM),
        ],
    )(xt, dinv, xs2, a0p, a1p, b, epnp, cdp)


def kernel(x, edge_attr, W, b, edge_index, time_steps=1):
    N = x.shape[0]
    E = edge_index.shape[1]
    M, L = 800, 125
    if M * L != N:
        M, L = N // 8, 8

    R = E // CH
    src2 = edge_index[0].reshape(R, CH)
    dst2 = edge_index[1].reshape(R, CH)
    attr1r = edge_attr[:, 1].reshape(R, CH)
    zn = jnp.zeros((N,), jnp.float32)

    deg_p, epn_p, cda_p = _sc_hist(src2, dst2, attr1r, zn)
    degp = deg_p.reshape(NC, M, L)
    epnp = epn_p.reshape(NC, M, L)
    cdp = cda_p.reshape(NC, M, L)

    xt0 = x.T.reshape(5, M, L)

    def body(_, carry):
        xt, vb, pp, bc, fr, dc = carry
        xs2, dinv = _tc_mid(xt, W, degp)
        xsf = xs2.reshape(2, N)
        a0_p, a1_p = _sc_msg(src2, dst2, xsf[0], xsf[1], zn)
        a0p = a0_p.reshape(NC, M, L)
        a1p = a1_p.reshape(NC, M, L)
        nxt, scal = _tc_final(xt, dinv, xs2, a0p, a1p, b, epnp, cdp)
        s = scal[0]
        return (nxt, vb + s[0:2], pp + s[2:4], bc + s[4], fr + s[5], dc + s[6])

    carry = (xt0, jnp.zeros((2,), jnp.float32), jnp.zeros((2,), jnp.float32),
             jnp.float32(0.0), jnp.float32(0.0), jnp.float32(0.0))
    xt, vb, pp, bc, fr, dc = lax.fori_loop(0, time_steps, body, carry)
    x_out = xt.reshape(5, N).T
    return (x_out, vb, pp, bc, fr, dc)


# speedup vs baseline: 231.2372x; 1.1635x over previous
import functools

import jax
import jax.numpy as jnp
from jax import lax
from jax.experimental import pallas as pl
from jax.experimental.pallas import tpu as pltpu
from jax.experimental.pallas import tpu_sc as plsc

ACC_SCALE = 0.02
MAX_VEL = 0.1

CH = 128
NC, NS = 2, 16
NW = NC * NS
CB = 64
G = 16


def _split(rows):
    main = (rows // (NW * CB)) * CB
    rem_total = rows - NW * main
    rem = rem_total // NW
    tail = rem_total - NW * rem
    return main, rem, tail


def _sc_hist(ei3, at3, zn):
    R = ei3.shape[0]
    N = zn.shape[0]
    MAIN, REM, TAIL = _split(R)
    NB = MAIN // CB

    mesh = plsc.VectorSubcoreMesh(core_axis_name="c", subcore_axis_name="s")

    @functools.partial(
        pl.kernel,
        out_type=[
            jax.ShapeDtypeStruct((NC, N), jnp.float32),
            jax.ShapeDtypeStruct((NC, N), jnp.float32),
            jax.ShapeDtypeStruct((NC, N), jnp.float32),
        ],
        mesh=mesh,
        compiler_params=pltpu.CompilerParams(use_tc_tiling_on_sc=False,
                                             needs_layout_passes=False),
        scratch_types=[
            pltpu.VMEM_SHARED((N,), jnp.float32),
            pltpu.VMEM_SHARED((N,), jnp.float32),
            pltpu.VMEM_SHARED((N,), jnp.float32),
            pltpu.VMEM((CB, 2, CH), jnp.int32),
            pltpu.VMEM((CB, 2, CH), jnp.float32),
            pltpu.VMEM((CH,), jnp.float32),
            pltpu.SemaphoreType.DMA,
        ],
    )
    def hist(ei_h, at_h, zn_h, deg_o, epn_o, cda_o,
             deg_s, epn_s, cda_s, ebuf, abuf, ones_v, sem):
        c = lax.axis_index("c")
        s = lax.axis_index("s")
        wid = s * NC + c

        for j in range(CH // 16):
            ones_v[pl.ds(j * 16, 16)] = jnp.ones((16,), jnp.float32)

        @pl.when(s == 0)
        def _():
            pltpu.sync_copy(zn_h, deg_s)

        @pl.when(s == 1)
        def _():
            pltpu.sync_copy(zn_h, epn_s)

        @pl.when(s == 2)
        def _():
            pltpu.sync_copy(zn_h, cda_s)

        plsc.subcore_barrier()

        def fire(k):
            pltpu.async_copy(ones_v, deg_s.at[ebuf.at[k, 1]], sem, add=True)
            pltpu.async_copy(ones_v, epn_s.at[ebuf.at[k, 0]], sem, add=True)
            pltpu.async_copy(abuf.at[k, 1], cda_s.at[ebuf.at[k, 0]], sem,
                             add=True)

        def drain(k):
            pltpu.make_async_copy(ones_v, deg_s.at[ebuf.at[k, 1]], sem).wait()
            pltpu.make_async_copy(ones_v, epn_s.at[ebuf.at[k, 0]], sem).wait()
            pltpu.make_async_copy(abuf.at[k, 1], cda_s.at[ebuf.at[k, 0]],
                                  sem).wait()

        def do_span(r, n):
            pltpu.sync_copy(ei_h.at[pl.ds(r, n)], ebuf.at[pl.ds(0, n)])
            pltpu.sync_copy(at_h.at[pl.ds(r, n)], abuf.at[pl.ds(0, n)])
            for k in range(n):
                fire(k)
            for k in range(n):
                drain(k)

        def blockm(blk, _):
            do_span(wid * MAIN + blk * CB, CB)
            return 0

        lax.fori_loop(0, NB, blockm, 0)

        if REM > 0:
            do_span(NW * MAIN + wid * REM, REM)

        if TAIL > 0:
            @pl.when((c == 0) & (s < TAIL))
            def _():
                do_span(NW * (MAIN + REM) + s, 1)

        plsc.subcore_barrier()

        @pl.when(s == 0)
        def _():
            pltpu.sync_copy(deg_s, deg_o.at[c])

        @pl.when(s == 1)
        def _():
            pltpu.sync_copy(epn_s, epn_o.at[c])

        @pl.when(s == 2)
        def _():
            pltpu.sync_copy(cda_s, cda_o.at[c])

    return hist(ei3, at3, zn)


def _sc_msg(ei3, xs0, xs1, zn):
    R = ei3.shape[0]
    N = xs0.shape[0]
    MAIN, REM, TAIL = _split(R)
    NB = MAIN // CB

    mesh = plsc.VectorSubcoreMesh(core_axis_name="c", subcore_axis_name="s")

    @functools.partial(
        pl.kernel,
        out_type=[
            jax.ShapeDtypeStruct((NC, N), jnp.float32),
            jax.ShapeDtypeStruct((NC, N), jnp.float32),
        ],
        mesh=mesh,
        compiler_params=pltpu.CompilerParams(use_tc_tiling_on_sc=False,
                                             needs_layout_passes=False),
        scratch_types=[
            pltpu.VMEM_SHARED((N,), jnp.float32),
            pltpu.VMEM_SHARED((N,), jnp.float32),
            pltpu.VMEM_SHARED((N,), jnp.float32),
            pltpu.VMEM_SHARED((N,), jnp.float32),
            pltpu.VMEM((CB, 2, CH), jnp.int32),
            pltpu.VMEM((CB, CH), jnp.float32),
            pltpu.VMEM((CB, CH), jnp.float32),
            pltpu.SemaphoreType.DMA,
            pltpu.SemaphoreType.DMA,
        ],
    )
    def msg(ei_h, xs0_h, xs1_h, zn_h, a0_o, a1_o,
            xs0_s, xs1_s, a0_s, a1_s, ebuf, m0, m1, gsem, ssem):
        c = lax.axis_index("c")
        s = lax.axis_index("s")
        wid = s * NC + c

        @pl.when(s == 0)
        def _():
            pltpu.sync_copy(xs0_h, xs0_s)

        @pl.when(s == 1)
        def _():
            pltpu.sync_copy(xs1_h, xs1_s)

        @pl.when(s == 2)
        def _():
            pltpu.sync_copy(zn_h, a0_s)

        @pl.when(s == 3)
        def _():
            pltpu.sync_copy(zn_h, a1_s)

        plsc.subcore_barrier()

        def fire_gather(k):
            pltpu.async_copy(xs0_s.at[ebuf.at[k, 0]], m0.at[k], gsem)
            pltpu.async_copy(xs1_s.at[ebuf.at[k, 0]], m1.at[k], gsem)

        def drain_gather(k):
            pltpu.make_async_copy(xs0_s.at[ebuf.at[k, 0]], m0.at[k],
                                  gsem).wait()
            pltpu.make_async_copy(xs1_s.at[ebuf.at[k, 0]], m1.at[k],
                                  gsem).wait()

        def fire_scatter(k):
            pltpu.async_copy(m0.at[k], a0_s.at[ebuf.at[k, 1]], ssem, add=True)
            pltpu.async_copy(m1.at[k], a1_s.at[ebuf.at[k, 1]], ssem, add=True)

        def drain_scatter(k):
            pltpu.make_async_copy(m0.at[k], a0_s.at[ebuf.at[k, 1]],
                                  ssem).wait()
            pltpu.make_async_copy(m1.at[k], a1_s.at[ebuf.at[k, 1]],
                                  ssem).wait()

        def do_span(r, n, gsz):
            pltpu.sync_copy(ei_h.at[pl.ds(r, n)], ebuf.at[pl.ds(0, n)])
            for g0 in range(0, n, gsz):
                gn = min(gsz, n - g0)
                for t in range(gn):
                    fire_gather(g0 + t)
                for t in range(gn):
                    drain_gather(g0 + t)
                for t in range(gn):
                    fire_scatter(g0 + t)
            for k in range(n):
                drain_scatter(k)

        def blockm(blk, _):
            do_span(wid * MAIN + blk * CB, CB, G)
            return 0

        lax.fori_loop(0, NB, blockm, 0)

        if REM > 0:
            do_span(NW * MAIN + wid * REM, REM, G)

        if TAIL > 0:
            @pl.when((c == 0) & (s < TAIL))
            def _():
                do_span(NW * (MAIN + REM) + s, 1, 1)

        plsc.subcore_barrier()

        @pl.when(s == 0)
        def _():
            pltpu.sync_copy(a0_s, a0_o.at[c])

        @pl.when(s == 1)
        def _():
            pltpu.sync_copy(a1_s, a1_o.at[c])

    return msg(ei3, xs0, xs1, zn)


def _tc_mid(xt, W, degp):
    _, M, L = xt.shape

    def body(xt_ref, w_ref, degp_ref, xs_ref, dinv_ref):
        deg = degp_ref[0] + degp_ref[1] + 1.0
        dinv = lax.rsqrt(deg)
        xw0 = (xt_ref[0] * w_ref[0, 0] + xt_ref[1] * w_ref[1, 0]
               + xt_ref[2] * w_ref[2, 0] + xt_ref[3] * w_ref[3, 0]
               + xt_ref[4] * w_ref[4, 0])
        xw1 = (xt_ref[0] * w_ref[0, 1] + xt_ref[1] * w_ref[1, 1]
               + xt_ref[2] * w_ref[2, 1] + xt_ref[3] * w_ref[3, 1]
               + xt_ref[4] * w_ref[4, 1])
        xs_ref[0] = xw0 * dinv
        xs_ref[1] = xw1 * dinv
        dinv_ref[...] = dinv

    return pl.pallas_call(
        body,
        out_shape=[
            jax.ShapeDtypeStruct((2, M, L), jnp.float32),
            jax.ShapeDtypeStruct((M, L), jnp.float32),
        ],
        in_specs=[
            pl.BlockSpec(memory_space=pltpu.VMEM),
            pl.BlockSpec(memory_space=pltpu.SMEM),
            pl.BlockSpec(memory_space=pltpu.VMEM),
        ],
    )(xt, W, degp)


def _tc_final(xt, dinv, xs2, a0p, a1p, b, epnp, cdp):
    _, M, L = xt.shape
    n_nodes = M * L

    def body(xt_ref, dinv_ref, xs_ref, a0_ref, a1_ref, b_ref, epnp_ref,
             cdp_ref, nx_ref, sc_ref):
        dinv = dinv_ref[...]
        x4 = xt_ref[4]
        acc0 = a0_ref[0] + a0_ref[1] + xs_ref[0]
        acc1 = a1_ref[0] + a1_ref[1] + xs_ref[1]
        g0 = dinv * acc0 + b_ref[0]
        g1 = dinv * acc1 + b_ref[1]
        food = (x4 == 1.0).astype(jnp.float32)
        a0 = g0 * ACC_SCALE * food
        a1 = g1 * ACC_SCALE * food
        v0 = jnp.clip(xt_ref[2] + a0, -MAX_VEL, MAX_VEL)
        v1 = jnp.clip(xt_ref[3] + a1, -MAX_VEL, MAX_VEL)
        p0 = xt_ref[0] + v0
        p1 = xt_ref[1] + v1
        nx_ref[0] = p0
        nx_ref[1] = p1
        nx_ref[2] = v0
        nx_ref[3] = v1
        nx_ref[4] = x4

        inv_n = jnp.float32(1.0 / n_nodes)
        vb0 = jnp.sum(jnp.abs(v0)) * inv_n
        vb1 = jnp.sum(jnp.abs(v1)) * inv_n
        pp0 = jnp.sum(jnp.abs(p0)) * inv_n
        pp1 = jnp.sum(jnp.abs(p1)) * inv_n
        ap0 = jnp.abs(p0)
        ap1 = jnp.abs(p1)
        border = (jnp.sum(jnp.log(ap0 + 1e-12) * (ap0 > 1.0))
                  + jnp.sum(jnp.log(ap1 + 1e-12) * (ap1 > 1.0)))
        epn = epnp_ref[0] + epnp_ref[1]
        fr = jnp.sum(((epn > 4.0) & (x4 == 0.0)).astype(jnp.float32))
        cd = cdp_ref[0] + cdp_ref[1]
        dc = jnp.sum(((x4 == 1.0) & (cd == 0.0)).astype(jnp.float32))

        idx = lax.broadcasted_iota(jnp.int32, (1, 128), 1)
        row = (jnp.where(idx == 0, vb0, 0.0) + jnp.where(idx == 1, vb1, 0.0)
               + jnp.where(idx == 2, pp0, 0.0) + jnp.where(idx == 3, pp1, 0.0)
               + jnp.where(idx == 4, border, 0.0)
               + jnp.where(idx == 5, fr, 0.0) + jnp.where(idx == 6, dc, 0.0))
        sc_ref[...] = row

    return pl.pallas_call(
        body,
        out_shape=[
            jax.ShapeDtypeStruct((5, M, L), jnp.float32),
            jax.ShapeDtypeStruct((1, 128), jnp.float32),
        ],
        in_specs=[
            pl.BlockSpec(memory_space=pltpu.VMEM),
            pl.BlockSpec(memory_space=pltpu.VMEM),
            pl.BlockSpec(memory_space=pltpu.VMEM),
            pl.BlockSpec(memory_space=pltpu.VMEM),
            pl.BlockSpec(memory_space=pltpu.VMEM),
            pl.BlockSpec(memory_space=pltpu.SMEM),
            pl.BlockSpec(memory_space=pltpu.VMEM),
            pl.BlockSpec(memory_space=pltpu.VMEM),
        ],
    )(xt, dinv, xs2, a0p, a1p, b, epnp, cdp)


def kernel(x, edge_attr, W, b, edge_index, time_steps=1):
    N = x.shape[0]
    E = edge_index.shape[1]
    M, L = 800, 125
    if M * L != N:
        M, L = N // 8, 8

    R = E // CH
    ei3 = jnp.transpose(edge_index.reshape(2, R, CH), (1, 0, 2))
    at3 = jnp.transpose(edge_attr.reshape(R, CH, 2), (0, 2, 1))
    zn = jnp.zeros((N,), jnp.float32)

    deg_p, epn_p, cda_p = _sc_hist(ei3, at3, zn)
    degp = deg_p.reshape(NC, M, L)
    epnp = epn_p.reshape(NC, M, L)
    cdp = cda_p.reshape(NC, M, L)

    xt0 = x.T.reshape(5, M, L)

    def body(_, carry):
        xt, vb, pp, bc, fr, dc = carry
        xs2, dinv = _tc_mid(xt, W, degp)
        xsf = xs2.reshape(2, N)
        a0_p, a1_p = _sc_msg(ei3, xsf[0], xsf[1], zn)
        a0p = a0_p.reshape(NC, M, L)
        a1p = a1_p.reshape(NC, M, L)
        nxt, scal = _tc_final(xt, dinv, xs2, a0p, a1p, b, epnp, cdp)
        s = scal[0]
        return (nxt, vb + s[0:2], pp + s[2:4], bc + s[4], fr + s[5], dc + s[6])

    carry = (xt0, jnp.zeros((2,), jnp.float32), jnp.zeros((2,), jnp.float32),
             jnp.float32(0.0), jnp.float32(0.0), jnp.float32(0.0))
    xt, vb, pp, bc, fr, dc = lax.fori_loop(0, time_steps, body, carry)
    x_out = xt.reshape(5, N).T
    return (x_out, vb, pp, bc, fr, dc)


# speedup vs baseline: 254.6583x; 1.1013x over previous
import functools

import jax
import jax.numpy as jnp
from jax import lax
from jax.experimental import pallas as pl
from jax.experimental.pallas import tpu as pltpu
from jax.experimental.pallas import tpu_sc as plsc

ACC_SCALE = 0.02
MAX_VEL = 0.1

CH = 128
NC, NS = 2, 16
NW = NC * NS
CB = 128
G = 32


def _split(rows):
    main = (rows // (NW * CB)) * CB
    rem_total = rows - NW * main
    rem = rem_total // NW
    tail = rem_total - NW * rem
    return main, rem, tail


def _sc_hist(ei3, at3, zn):
    R = ei3.shape[0]
    N = zn.shape[0]
    MAIN, REM, TAIL = _split(R)
    NB = MAIN // CB

    mesh = plsc.VectorSubcoreMesh(core_axis_name="c", subcore_axis_name="s")

    @functools.partial(
        pl.kernel,
        out_type=[
            jax.ShapeDtypeStruct((NC, N), jnp.float32),
            jax.ShapeDtypeStruct((NC, N), jnp.float32),
            jax.ShapeDtypeStruct((NC, N), jnp.float32),
        ],
        mesh=mesh,
        compiler_params=pltpu.CompilerParams(use_tc_tiling_on_sc=False,
                                             needs_layout_passes=False),
        scratch_types=[
            pltpu.VMEM_SHARED((N,), jnp.float32),
            pltpu.VMEM_SHARED((N,), jnp.float32),
            pltpu.VMEM_SHARED((N,), jnp.float32),
            pltpu.VMEM((CB, 2, CH), jnp.int32),
            pltpu.VMEM((CB, 2, CH), jnp.float32),
            pltpu.VMEM((CH,), jnp.float32),
            pltpu.SemaphoreType.DMA,
        ],
    )
    def hist(ei_h, at_h, zn_h, deg_o, epn_o, cda_o,
             deg_s, epn_s, cda_s, ebuf, abuf, ones_v, sem):
        c = lax.axis_index("c")
        s = lax.axis_index("s")
        wid = s * NC + c

        for j in range(CH // 16):
            ones_v[pl.ds(j * 16, 16)] = jnp.ones((16,), jnp.float32)

        @pl.when(s == 0)
        def _():
            pltpu.sync_copy(zn_h, deg_s)

        @pl.when(s == 1)
        def _():
            pltpu.sync_copy(zn_h, epn_s)

        @pl.when(s == 2)
        def _():
            pltpu.sync_copy(zn_h, cda_s)

        plsc.subcore_barrier()

        def fire(k):
            pltpu.async_copy(ones_v, deg_s.at[ebuf.at[k, 1]], sem, add=True)
            pltpu.async_copy(ones_v, epn_s.at[ebuf.at[k, 0]], sem, add=True)
            pltpu.async_copy(abuf.at[k, 1], cda_s.at[ebuf.at[k, 0]], sem,
                             add=True)

        def drain(k):
            pltpu.make_async_copy(ones_v, deg_s.at[ebuf.at[k, 1]], sem).wait()
            pltpu.make_async_copy(ones_v, epn_s.at[ebuf.at[k, 0]], sem).wait()
            pltpu.make_async_copy(abuf.at[k, 1], cda_s.at[ebuf.at[k, 0]],
                                  sem).wait()

        def do_span(r, n):
            pltpu.sync_copy(ei_h.at[pl.ds(r, n)], ebuf.at[pl.ds(0, n)])
            pltpu.sync_copy(at_h.at[pl.ds(r, n)], abuf.at[pl.ds(0, n)])
            for k in range(n):
                fire(k)
            for k in range(n):
                drain(k)

        def blockm(blk, _):
            do_span(wid * MAIN + blk * CB, CB)
            return 0

        lax.fori_loop(0, NB, blockm, 0)

        if REM > 0:
            do_span(NW * MAIN + wid * REM, REM)

        if TAIL > 0:
            @pl.when((c == 0) & (s < TAIL))
            def _():
                do_span(NW * (MAIN + REM) + s, 1)

        plsc.subcore_barrier()

        @pl.when(s == 0)
        def _():
            pltpu.sync_copy(deg_s, deg_o.at[c])

        @pl.when(s == 1)
        def _():
            pltpu.sync_copy(epn_s, epn_o.at[c])

        @pl.when(s == 2)
        def _():
            pltpu.sync_copy(cda_s, cda_o.at[c])

    return hist(ei3, at3, zn)


def _sc_msg(ei3, xs0, xs1, zn):
    R = ei3.shape[0]
    N = xs0.shape[0]
    MAIN, REM, TAIL = _split(R)
    NB = MAIN // CB

    mesh = plsc.VectorSubcoreMesh(core_axis_name="c", subcore_axis_name="s")

    @functools.partial(
        pl.kernel,
        out_type=[
            jax.ShapeDtypeStruct((NC, N), jnp.float32),
            jax.ShapeDtypeStruct((NC, N), jnp.float32),
        ],
        mesh=mesh,
        compiler_params=pltpu.CompilerParams(use_tc_tiling_on_sc=False,
                                             needs_layout_passes=False),
        scratch_types=[
            pltpu.VMEM_SHARED((N,), jnp.float32),
            pltpu.VMEM_SHARED((N,), jnp.float32),
            pltpu.VMEM_SHARED((N,), jnp.float32),
            pltpu.VMEM_SHARED((N,), jnp.float32),
            pltpu.VMEM((CB, 2, CH), jnp.int32),
            pltpu.VMEM((CB, CH), jnp.float32),
            pltpu.VMEM((CB, CH), jnp.float32),
            pltpu.SemaphoreType.DMA,
            pltpu.SemaphoreType.DMA,
        ],
    )
    def msg(ei_h, xs0_h, xs1_h, zn_h, a0_o, a1_o,
            xs0_s, xs1_s, a0_s, a1_s, ebuf, m0, m1, gsem, ssem):
        c = lax.axis_index("c")
        s = lax.axis_index("s")
        wid = s * NC + c

        @pl.when(s == 0)
        def _():
            pltpu.sync_copy(xs0_h, xs0_s)

        @pl.when(s == 1)
        def _():
            pltpu.sync_copy(xs1_h, xs1_s)

        @pl.when(s == 2)
        def _():
            pltpu.sync_copy(zn_h, a0_s)

        @pl.when(s == 3)
        def _():
            pltpu.sync_copy(zn_h, a1_s)

        plsc.subcore_barrier()

        def fire_gather(k):
            pltpu.async_copy(xs0_s.at[ebuf.at[k, 0]], m0.at[k], gsem)
            pltpu.async_copy(xs1_s.at[ebuf.at[k, 0]], m1.at[k], gsem)

        def drain_gather(k):
            pltpu.make_async_copy(xs0_s.at[ebuf.at[k, 0]], m0.at[k],
                                  gsem).wait()
            pltpu.make_async_copy(xs1_s.at[ebuf.at[k, 0]], m1.at[k],
                                  gsem).wait()

        def fire_scatter(k):
            pltpu.async_copy(m0.at[k], a0_s.at[ebuf.at[k, 1]], ssem, add=True)
            pltpu.async_copy(m1.at[k], a1_s.at[ebuf.at[k, 1]], ssem, add=True)

        def drain_scatter(k):
            pltpu.make_async_copy(m0.at[k], a0_s.at[ebuf.at[k, 1]],
                                  ssem).wait()
            pltpu.make_async_copy(m1.at[k], a1_s.at[ebuf.at[k, 1]],
                                  ssem).wait()

        def do_span(r, n, gsz):
            pltpu.sync_copy(ei_h.at[pl.ds(r, n)], ebuf.at[pl.ds(0, n)])
            for g0 in range(0, n, gsz):
                gn = min(gsz, n - g0)
                for t in range(gn):
                    fire_gather(g0 + t)
                for t in range(gn):
                    drain_gather(g0 + t)
                for t in range(gn):
                    fire_scatter(g0 + t)
            for k in range(n):
                drain_scatter(k)

        def blockm(blk, _):
            do_span(wid * MAIN + blk * CB, CB, G)
            return 0

        lax.fori_loop(0, NB, blockm, 0)

        if REM > 0:
            do_span(NW * MAIN + wid * REM, REM, G)

        if TAIL > 0:
            @pl.when((c == 0) & (s < TAIL))
            def _():
                do_span(NW * (MAIN + REM) + s, 1, 1)

        plsc.subcore_barrier()

        @pl.when(s == 0)
        def _():
            pltpu.sync_copy(a0_s, a0_o.at[c])

        @pl.when(s == 1)
        def _():
            pltpu.sync_copy(a1_s, a1_o.at[c])

    return msg(ei3, xs0, xs1, zn)


def _tc_mid(xt, W, degp):
    _, M, L = xt.shape

    def body(xt_ref, w_ref, degp_ref, xs_ref, dinv_ref):
        deg = degp_ref[0] + degp_ref[1] + 1.0
        dinv = lax.rsqrt(deg)
        xw0 = (xt_ref[0] * w_ref[0, 0] + xt_ref[1] * w_ref[1, 0]
               + xt_ref[2] * w_ref[2, 0] + xt_ref[3] * w_ref[3, 0]
               + xt_ref[4] * w_ref[4, 0])
        xw1 = (xt_ref[0] * w_ref[0, 1] + xt_ref[1] * w_ref[1, 1]
               + xt_ref[2] * w_ref[2, 1] + xt_ref[3] * w_ref[3, 1]
               + xt_ref[4] * w_ref[4, 1])
        xs_ref[0] = xw0 * dinv
        xs_ref[1] = xw1 * dinv
        dinv_ref[...] = dinv

    return pl.pallas_call(
        body,
        out_shape=[
            jax.ShapeDtypeStruct((2, M, L), jnp.float32),
            jax.ShapeDtypeStruct((M, L), jnp.float32),
        ],
        in_specs=[
            pl.BlockSpec(memory_space=pltpu.VMEM),
            pl.BlockSpec(memory_space=pltpu.SMEM),
            pl.BlockSpec(memory_space=pltpu.VMEM),
        ],
    )(xt, W, degp)


def _tc_final(xt, dinv, xs2, a0p, a1p, b, epnp, cdp):
    _, M, L = xt.shape
    n_nodes = M * L

    def body(xt_ref, dinv_ref, xs_ref, a0_ref, a1_ref, b_ref, epnp_ref,
             cdp_ref, nx_ref, sc_ref):
        dinv = dinv_ref[...]
        x4 = xt_ref[4]
        acc0 = a0_ref[0] + a0_ref[1] + xs_ref[0]
        acc1 = a1_ref[0] + a1_ref[1] + xs_ref[1]
        g0 = dinv * acc0 + b_ref[0]
        g1 = dinv * acc1 + b_ref[1]
        food = (x4 == 1.0).astype(jnp.float32)
        a0 = g0 * ACC_SCALE * food
        a1 = g1 * ACC_SCALE * food
        v0 = jnp.clip(xt_ref[2] + a0, -MAX_VEL, MAX_VEL)
        v1 = jnp.clip(xt_ref[3] + a1, -MAX_VEL, MAX_VEL)
        p0 = xt_ref[0] + v0
        p1 = xt_ref[1] + v1
        nx_ref[0] = p0
        nx_ref[1] = p1
        nx_ref[2] = v0
        nx_ref[3] = v1
        nx_ref[4] = x4

        inv_n = jnp.float32(1.0 / n_nodes)
        vb0 = jnp.sum(jnp.abs(v0)) * inv_n
        vb1 = jnp.sum(jnp.abs(v1)) * inv_n
        pp0 = jnp.sum(jnp.abs(p0)) * inv_n
        pp1 = jnp.sum(jnp.abs(p1)) * inv_n
        ap0 = jnp.abs(p0)
        ap1 = jnp.abs(p1)
        border = (jnp.sum(jnp.log(ap0 + 1e-12) * (ap0 > 1.0))
                  + jnp.sum(jnp.log(ap1 + 1e-12) * (ap1 > 1.0)))
        epn = epnp_ref[0] + epnp_ref[1]
        fr = jnp.sum(((epn > 4.0) & (x4 == 0.0)).astype(jnp.float32))
        cd = cdp_ref[0] + cdp_ref[1]
        dc = jnp.sum(((x4 == 1.0) & (cd == 0.0)).astype(jnp.float32))

        idx = lax.broadcasted_iota(jnp.int32, (1, 128), 1)
        row = (jnp.where(idx == 0, vb0, 0.0) + jnp.where(idx == 1, vb1, 0.0)
               + jnp.where(idx == 2, pp0, 0.0) + jnp.where(idx == 3, pp1, 0.0)
               + jnp.where(idx == 4, border, 0.0)
               + jnp.where(idx == 5, fr, 0.0) + jnp.where(idx == 6, dc, 0.0))
        sc_ref[...] = row

    return pl.pallas_call(
        body,
        out_shape=[
            jax.ShapeDtypeStruct((5, M, L), jnp.float32),
            jax.ShapeDtypeStruct((1, 128), jnp.float32),
        ],
        in_specs=[
            pl.BlockSpec(memory_space=pltpu.VMEM),
            pl.BlockSpec(memory_space=pltpu.VMEM),
            pl.BlockSpec(memory_space=pltpu.VMEM),
            pl.BlockSpec(memory_space=pltpu.VMEM),
            pl.BlockSpec(memory_space=pltpu.VMEM),
            pl.BlockSpec(memory_space=pltpu.SMEM),
            pl.BlockSpec(memory_space=pltpu.VMEM),
            pl.BlockSpec(memory_space=pltpu.VMEM),
        ],
    )(xt, dinv, xs2, a0p, a1p, b, epnp, cdp)


def kernel(x, edge_attr, W, b, edge_index, time_steps=1):
    N = x.shape[0]
    E = edge_index.shape[1]
    M, L = 800, 125
    if M * L != N:
        M, L = N // 8, 8

    R = E // CH
    ei3 = jnp.transpose(edge_index.reshape(2, R, CH), (1, 0, 2))
    at3 = jnp.transpose(edge_attr.reshape(R, CH, 2), (0, 2, 1))
    zn = jnp.zeros((N,), jnp.float32)

    deg_p, epn_p, cda_p = _sc_hist(ei3, at3, zn)
    degp = deg_p.reshape(NC, M, L)
    epnp = epn_p.reshape(NC, M, L)
    cdp = cda_p.reshape(NC, M, L)

    xt0 = x.T.reshape(5, M, L)

    def body(_, carry):
        xt, vb, pp, bc, fr, dc = carry
        xs2, dinv = _tc_mid(xt, W, degp)
        xsf = xs2.reshape(2, N)
        a0_p, a1_p = _sc_msg(ei3, xsf[0], xsf[1], zn)
        a0p = a0_p.reshape(NC, M, L)
        a1p = a1_p.reshape(NC, M, L)
        nxt, scal = _tc_final(xt, dinv, xs2, a0p, a1p, b, epnp, cdp)
        s = scal[0]
        return (nxt, vb + s[0:2], pp + s[2:4], bc + s[4], fr + s[5], dc + s[6])

    carry = (xt0, jnp.zeros((2,), jnp.float32), jnp.zeros((2,), jnp.float32),
             jnp.float32(0.0), jnp.float32(0.0), jnp.float32(0.0))
    xt, vb, pp, bc, fr, dc = lax.fori_loop(0, time_steps, body, carry)
    x_out = xt.reshape(5, N).T
    return (x_out, vb, pp, bc, fr, dc)
